# baseline (speedup 1.0000x reference)
import functools

import jax
import jax.numpy as jnp
from jax import lax
from jax.experimental import pallas as pl
from jax.experimental.pallas import tpu as pltpu
from jax.experimental.pallas import tpu_sc as plsc

N = 4096
D = 512
E = 131072
TOPK = 8
K = 64
NSEEDS = 3
NITER = 10

def _i32(x):
    return jnp.asarray(x, jnp.int32)


ROWS_A = 256
ROWS_E = 256
NTILES = 32


def _simtopk_body(sn_ref, tnt_ref, knn_ref):
    i = pl.program_id(0)
    sim = jnp.dot(sn_ref[...], tnt_ref[...], preferred_element_type=jnp.float32)
    rows = i * ROWS_A + lax.broadcasted_iota(jnp.int32, (ROWS_A, N), 0)
    cols = lax.broadcasted_iota(jnp.int32, (ROWS_A, N), 1)
    sim = jnp.where(rows == cols, sim + 10.0, sim)
    vals = sim
    kcols = lax.broadcasted_iota(jnp.int32, (ROWS_A, TOPK), 1)
    out = jnp.zeros((ROWS_A, TOPK), jnp.int32)
    for k in range(TOPK):
        m = jnp.max(vals, axis=1, keepdims=True)
        am = jnp.min(jnp.where(vals == m, cols, N), axis=1)
        out = jnp.where(kcols == k, am[:, None], out)
        vals = jnp.where(cols == am[:, None], -jnp.inf, vals)
    knn_ref[...] = out


def _simtopk(sn, tnt):
    return pl.pallas_call(
        _simtopk_body,
        grid=(N // ROWS_A,),
        in_specs=[
            pl.BlockSpec((ROWS_A, D), lambda i: (i, _i32(0))),
            pl.BlockSpec((D, N), lambda i: (_i32(0), _i32(0))),
        ],
        out_specs=pl.BlockSpec((ROWS_A, TOPK), lambda i: (i, _i32(0))),
        out_shape=jax.ShapeDtypeStruct((N, TOPK), jnp.int32),
    )(sn, tnt)


def _assign_labels(d2):
    cols = lax.broadcasted_iota(jnp.int32, (N, K), 1)
    mn = jnp.min(d2, axis=1, keepdims=True)
    lab = jnp.min(jnp.where(d2 == mn, cols, K), axis=1)
    return lab, cols


def _assign_iter_body(tn_ref, dsq_ref, cent_t_ref, csq_ref, lab_ref, cnt_ref):
    data = tn_ref[...]
    prod = jnp.dot(data, cent_t_ref[...], preferred_element_type=jnp.float32)
    d2 = dsq_ref[...] - 2.0 * prod + csq_ref[...]
    lab, cols = _assign_labels(d2)
    oh = (lab[:, None] == cols).astype(jnp.float32)
    lab_ref[...] = lab[None, :]
    cnt_ref[...] = jnp.sum(oh, axis=0, keepdims=True)


def _assign_iter(tn, data_sq, cent_t, cent_sq):
    return pl.pallas_call(
        _assign_iter_body,
        out_shape=[
            jax.ShapeDtypeStruct((1, N), jnp.int32),
            jax.ShapeDtypeStruct((1, K), jnp.float32),
        ],
    )(tn, data_sq, cent_t, cent_sq)


def _assign_final_body(tn_ref, dsq_ref, cent_t_ref, csq_ref, oh_ref):
    data = tn_ref[...]
    prod = jnp.dot(data, cent_t_ref[...], preferred_element_type=jnp.float32)
    d2 = dsq_ref[...] - 2.0 * prod + csq_ref[...]
    lab, cols = _assign_labels(d2)
    oh_ref[...] = (lab[:, None] == cols).astype(jnp.float32)


def _assign_final(tn, data_sq, cent_t, cent_sq):
    return pl.pallas_call(
        _assign_final_body,
        out_shape=jax.ShapeDtypeStruct((N, K), jnp.float32),
    )(tn, data_sq, cent_t, cent_sq)


def _kmeans(tn, cent0):
    data_sq = jnp.sum(tn * tn, axis=1, keepdims=True)
    cents = [cent0[s] for s in range(NSEEDS)]
    for _ in range(NITER):
        for s in range(NSEEDS):
            cent = cents[s]
            lab2d, cnt2d = _assign_iter(tn, data_sq, cent.T,
                                        jnp.sum(cent * cent, axis=1)[None, :])
            labels = lab2d[0].astype(jnp.int64)
            sums = jax.ops.segment_sum(tn, labels, num_segments=K)
            cents[s] = sums / jnp.maximum(cnt2d[0], 1.0)[:, None]
    ohs = []
    for s in range(NSEEDS):
        cent = cents[s]
        ohs.append(_assign_final(tn, data_sq, cent.T,
                                 jnp.sum(cent * cent, axis=1)[None, :]))
    return jnp.stack(ohs)


NHALF = 2
NCHUNK = NTILES // NHALF
EPC = E // NCHUNK
HROWS = N // NHALF


def _edge_match_sc(r_arr, c_arr, knn_flat):
    mesh = plsc.VectorSubcoreMesh(core_axis_name="c", subcore_axis_name="s")

    @functools.partial(
        pl.kernel,
        mesh=mesh,
        out_type=jax.ShapeDtypeStruct(
            (NHALF, NCHUNK, HROWS * TOPK // 16, 16), jnp.float32),
        compiler_params=pltpu.CompilerParams(needs_layout_passes=False,
                                             use_tc_tiling_on_sc=False),
        scratch_types=[
            pltpu.VMEM((HROWS * TOPK // 16, 16), jnp.int32),
            pltpu.VMEM((HROWS * TOPK // 16, 16), jnp.float32),
            pltpu.VMEM((EPC,), jnp.int32),
            pltpu.VMEM((EPC,), jnp.int32),
        ],
    )
    def kern(r_hbm, c_hbm, knn_hbm, out_hbm, knn_v, acc_v, r_v, c_v):
        two = jnp.int32(2)
        wid = lax.axis_index("s").astype(jnp.int32) * two + lax.axis_index(
            "c").astype(jnp.int32)
        half = wid & jnp.int32(1)
        echunk = lax.shift_right_logical(wid, jnp.int32(1))
        pltpu.sync_copy(knn_hbm.at[half], knn_v)
        pltpu.sync_copy(r_hbm.at[echunk], r_v)
        pltpu.sync_copy(c_hbm.at[echunk], c_v)

        zeros16 = jnp.zeros((16,), jnp.float32)

        def zbody(i):
            acc_v[i.astype(jnp.int32), :] = zeros16

        pl.loop(jnp.int32(0), jnp.int32(HROWS * TOPK // 16))(zbody)

        ones16 = jnp.ones((16,), jnp.float32)
        eight = jnp.int32(8)
        fifteen = jnp.int32(15)
        rbase = half * jnp.int32(HROWS)

        def body(g):
            base = g.astype(jnp.int32) * 16
            r = r_v[pl.ds(base, 16)]
            c = c_v[pl.ds(base, 16)]
            lr = r - rbase
            valid = (lr >= 0) & (lr < HROWS)
            lr = jnp.where(valid, lr, 0)
            r8 = lr * eight
            for k in range(TOPK):
                idx = r8 + jnp.int32(k)
                hi = lax.shift_right_logical(idx, jnp.int32(4))
                lo = idx & fifteen
                vals = plsc.load_gather(knn_v, [hi, lo], mask=valid)
                m = valid & (vals == c)
                plsc.addupdate_scatter(acc_v, [hi, lo], ones16, mask=m)

        pl.loop(jnp.int32(0), jnp.int32(EPC // 16))(body)

        pltpu.sync_copy(acc_v, out_hbm.at[half, echunk])

    return kern(r_arr, c_arr, knn_flat)


def _assemble_body(knn_ref, part_ref, m_ref, pos_ref):
    adj = jnp.sum(part_ref[0], axis=0)
    i = pl.program_id(0)
    s = jnp.zeros((ROWS_E, N), jnp.float32)
    for r in range(NSEEDS):
        mblk = m_ref[r, pl.ds(i * ROWS_E, ROWS_E)].astype(jnp.bfloat16)
        s = s + lax.dot_general(mblk, m_ref[r].astype(jnp.bfloat16),
                                (((1,), (1,)), ((), ())),
                                preferred_element_type=jnp.float32)
    g = (s >= 0.5).astype(jnp.float32)
    cols = lax.broadcasted_iota(jnp.int32, (ROWS_E, N), 1)
    knn = knn_ref[...]
    pos = jnp.zeros((ROWS_E, N), jnp.float32)
    for k in range(TOPK):
        mask = cols == knn[:, k][:, None]
        pos = pos + jnp.where(mask, adj[:, k][:, None] + g, 0.0)
    pos_ref[...] = pos


def _assemble(knn, part, m):
    return pl.pallas_call(
        _assemble_body,
        grid=(N // ROWS_E,),
        in_specs=[
            pl.BlockSpec((ROWS_E, TOPK), lambda i: (i, _i32(0))),
            pl.BlockSpec((1, NCHUNK, ROWS_E, TOPK),
                         lambda i: (i // (HROWS // ROWS_E), _i32(0),
                                    i % (HROWS // ROWS_E), _i32(0))),
            pl.BlockSpec((NSEEDS, N, K), lambda i: (_i32(0), _i32(0), _i32(0))),
        ],
        out_specs=pl.BlockSpec((ROWS_E, N), lambda i: (i, _i32(0))),
        out_shape=jax.ShapeDtypeStruct((N, N), jnp.float32),
    )(knn, part, m)


def kernel(student, teacher, edge_index, top_k, epoch):
    student = student / jnp.linalg.norm(student, axis=-1, keepdims=True)
    teacher = teacher / jnp.linalg.norm(teacher, axis=-1, keepdims=True)
    sn = student
    tn = jax.lax.stop_gradient(teacher)

    knn = _simtopk(sn, tn.T)

    cent0 = jnp.stack([
        tn[jax.random.choice(jax.random.key(1234 + s), N, shape=(K,),
                             replace=False)]
        for s in range(NSEEDS)
    ])
    m = _kmeans(tn, cent0)

    er = edge_index[0].astype(jnp.int32).reshape(NCHUNK, EPC)
    ec = edge_index[1].astype(jnp.int32).reshape(NCHUNK, EPC)
    part = _edge_match_sc(er, ec,
                          knn.reshape(NHALF, HROWS * TOPK // 16, 16))

    pos = _assemble(knn, part.reshape(NHALF, NCHUNK, HROWS, TOPK), m)
    return pos, knn + (top_k - TOPK)
